# Initial kernel scaffold; baseline (speedup 1.0000x reference)
import functools

import jax
import jax.numpy as jnp
from jax import lax
from jax.experimental import pallas as pl
from jax.experimental.pallas import tpu as pltpu
from jax.experimental.pallas import tpu_sc as plsc

N_LEVELS = 16
HASHMAP_SIZE = 2 ** 19
GRID_SIZE = 512.0
MASK = HASHMAP_SIZE - 1
P2 = -1640531535
P3 = 805459861

N_POINTS = 262144
D_OUT = 2 * N_LEVELS
NW = 32
BPW = N_POINTS // NW
CHUNK = 1024
GCH = 128
L = 16


def _sc_encode(xT, tabT):
    mesh = plsc.VectorSubcoreMesh(core_axis_name="c", subcore_axis_name="s")

    @functools.partial(
        pl.kernel,
        mesh=mesh,
        out_type=jax.ShapeDtypeStruct((N_POINTS, D_OUT), jnp.float32),
        scratch_types=[
            pltpu.VMEM((3, BPW), jnp.float32),
            pltpu.VMEM((BPW,), jnp.int32),
            pltpu.VMEM((CHUNK, D_OUT), jnp.float32),
            pltpu.SemaphoreType.DMA,
        ],
    )
    def k(xT_hbm, tabT_hbm, out_hbm, xyz_v, idx_v, rows_v, sem):
        wid = lax.axis_index("s") * 2 + lax.axis_index("c")
        base = wid * BPW
        pltpu.sync_copy(xT_hbm.at[:, pl.ds(base, BPW)], xyz_v)

        def hash_body(i, carry):
            s = i * L
            fx = xyz_v[0, pl.ds(s, L)]
            fy = xyz_v[1, pl.ds(s, L)]
            fz = xyz_v[2, pl.ds(s, L)]
            cx = jnp.floor(fx * GRID_SIZE).astype(jnp.int32)
            cy = jnp.floor(fy * GRID_SIZE).astype(jnp.int32)
            cz = jnp.floor(fz * GRID_SIZE).astype(jnp.int32)
            h = (cx ^ (cy * jnp.int32(P2)) ^ (cz * jnp.int32(P3))) & jnp.int32(MASK)
            idx_v[pl.ds(s, L)] = h
            return carry

        lax.fori_loop(0, BPW // L, hash_body, 0)

        def chunk_body(j, carry):
            cps = []
            for t in range(CHUNK // GCH):
                cps.append(pltpu.async_copy(
                    tabT_hbm.at[idx_v.at[pl.ds(j * CHUNK + t * GCH, GCH)]],
                    rows_v.at[pl.ds(t * GCH, GCH)],
                    sem,
                ))
            for c in cps:
                c.wait()
            pltpu.sync_copy(rows_v, out_hbm.at[pl.ds(base + j * CHUNK, CHUNK)])
            return carry

        lax.fori_loop(0, BPW // CHUNK, chunk_body, 0)

    return k(xT, tabT)


def kernel(x, tables):
    xT = x.T
    tabT = jnp.transpose(tables, (1, 0, 2)).reshape(HASHMAP_SIZE, D_OUT)
    return _sc_encode(xT, tabT)


# speedup vs baseline: 1.3673x; 1.3673x over previous
import functools

import jax
import jax.numpy as jnp
from jax import lax
from jax.experimental import pallas as pl
from jax.experimental.pallas import tpu as pltpu
from jax.experimental.pallas import tpu_sc as plsc

N_LEVELS = 16
HASHMAP_SIZE = 2 ** 19
GRID_SIZE = 512.0
MASK = HASHMAP_SIZE - 1
P2 = -1640531535
P3 = 805459861

N_POINTS = 262144
D_OUT = 2 * N_LEVELS
NW = 32
BPW = N_POINTS // NW
CHUNK = 1024
GCH = 128
L = 16


def _sc_encode(xT, tabT):
    mesh = plsc.VectorSubcoreMesh(core_axis_name="c", subcore_axis_name="s")

    @functools.partial(
        pl.kernel,
        mesh=mesh,
        out_type=jax.ShapeDtypeStruct((N_POINTS, D_OUT), jnp.float32),
        compiler_params=pltpu.CompilerParams(use_tc_tiling_on_sc=False),
        scratch_types=[
            pltpu.VMEM((3, BPW), jnp.float32),
            pltpu.VMEM((BPW,), jnp.int32),
            pltpu.VMEM((CHUNK, D_OUT), jnp.float32),
            pltpu.SemaphoreType.DMA,
        ],
    )
    def k(xT_hbm, tabT_hbm, out_hbm, xyz_v, idx_v, rows_v, sem):
        wid = lax.axis_index("s") * jnp.int32(2) + lax.axis_index("c")
        base = wid * jnp.int32(BPW)
        pltpu.sync_copy(xT_hbm.at[:, pl.ds(base, BPW)], xyz_v)

        def hash_body(i, carry):
            s = i * jnp.int32(L)
            fx = xyz_v[0, pl.ds(s, L)]
            fy = xyz_v[1, pl.ds(s, L)]
            fz = xyz_v[2, pl.ds(s, L)]
            cx = (fx * GRID_SIZE).astype(jnp.int32)
            cy = (fy * GRID_SIZE).astype(jnp.int32)
            cz = (fz * GRID_SIZE).astype(jnp.int32)
            h = (cx ^ (cy * jnp.int32(P2)) ^ (cz * jnp.int32(P3))) & jnp.int32(MASK)
            idx_v[pl.ds(s, L)] = h
            return carry

        lax.fori_loop(jnp.int32(0), jnp.int32(BPW // L), hash_body, jnp.int32(0))

        def chunk_body(j, carry):
            off = j * jnp.int32(CHUNK)
            cps = []
            for t in range(CHUNK // GCH):
                cps.append(pltpu.async_copy(
                    tabT_hbm.at[idx_v.at[pl.ds(off + jnp.int32(t * GCH), GCH)]],
                    rows_v.at[pl.ds(t * GCH, GCH)],
                    sem,
                ))
            for c in cps:
                c.wait()
            pltpu.sync_copy(rows_v, out_hbm.at[pl.ds(base + off, CHUNK)])
            return carry

        lax.fori_loop(jnp.int32(0), jnp.int32(BPW // CHUNK), chunk_body, jnp.int32(0))

    return k(xT, tabT)


def kernel(x, tables):
    xT = x.T
    tabT = jnp.transpose(tables, (1, 0, 2)).reshape(HASHMAP_SIZE, D_OUT)
    return _sc_encode(xT, tabT)
